# baseline (speedup 1.0000x reference)
import functools

import jax
import jax.numpy as jnp
from jax import lax
from jax.experimental import pallas as pl
from jax.experimental.pallas import tpu as pltpu
from jax.experimental.pallas import tpu_sc as plsc

N_BREAKS = 128
INPUT_RANGE = 8.0
DX = 2.0 * INPUT_RANGE / (N_BREAKS - 1)

NUM_CORES = 2
NUM_SUBCORES = 16
NUM_WORKERS = NUM_CORES * NUM_SUBCORES
LANES = 16


def _table_body(w_ref, out_ref):
    n = N_BREAKS
    R = INPUT_RANGE
    dx = DX
    f32 = jnp.float32
    hi = lax.Precision.HIGHEST

    w = w_ref[...]
    ri = lax.broadcasted_iota(jnp.int32, (n, n), 0)
    ci = lax.broadcasted_iota(jnp.int32, (n, n), 1)

    sp = jnp.maximum(w, 0.0) + jnp.log(1.0 + jnp.exp(-jnp.abs(w)))
    cell = sp * (dx * dx)

    U = (ri <= ci).astype(f32)
    L = (ci <= ri).astype(f32)
    cx = lax.dot(cell, U, precision=hi)
    cxy = lax.dot(L, cx, precision=hi)

    D1 = (ci == ri - 1).astype(f32)
    M1 = (ri == ci - 1).astype(f32)
    interior = lax.dot(D1, lax.dot(cxy, M1, precision=hi), precision=hi)

    row0 = jnp.sum(jnp.where(ri == 0, cx, 0.0), axis=0, keepdims=True) * (1.0 / dx)
    row0s = lax.dot(jnp.broadcast_to(row0, (n, n)), M1, precision=hi)
    cy = lax.dot(L, cell, precision=hi)
    col0 = jnp.sum(jnp.where(ci == 0, cy, 0.0), axis=1, keepdims=True) * (1.0 / dx)
    col0s = lax.dot(D1, jnp.broadcast_to(col0, (n, n)), precision=hi)

    res = jnp.where(ri == 0, row0s, jnp.where(ci == 0, col0s, interior)) - R

    tr = jnp.sum(jnp.where((ri == n - 1) & (ci == n - 1), res, 0.0)) + R
    ok = tr > 1e-10
    scale = (2.0 * R) / jnp.where(ok, tr, 1.0)
    res = jnp.where(ok, -R + (res + R) * scale, res)

    a = res + R
    for k in (1, 2, 4, 8, 16, 32, 64):
        Dk = (ci == ri - k).astype(f32)
        a = jnp.maximum(a, lax.dot(Dk, a, precision=hi))
    for k in (1, 2, 4, 8, 16, 32, 64):
        Mk = (ri == ci - k).astype(f32)
        a = jnp.maximum(a, lax.dot(a, Mk, precision=hi))
    out_ref[...] = a - R


def _build_table(w):
    return pl.pallas_call(
        _table_body,
        out_shape=jax.ShapeDtypeStruct((N_BREAKS, N_BREAKS), jnp.float32),
    )(w)


def _interp_vec(xx, yy, tab_v):
    n = N_BREAKS
    inv_dx = float(1.0 / DX)

    xn = (xx + INPUT_RANGE) * inv_dx
    yn = (yy + INPUT_RANGE) * inv_dx
    xc = jnp.clip(xn, 0.0, float(n - 1))
    yc = jnp.clip(yn, 0.0, float(n - 1))
    ix = xc.astype(jnp.int32)
    iy = yc.astype(jnp.int32)
    alpha = jnp.clip(xn - ix.astype(jnp.float32), 0.0, 1.0)
    beta = jnp.clip(yn - iy.astype(jnp.float32), 0.0, 1.0)
    ixn = jnp.minimum(ix + 1, n - 1)
    iyn = jnp.minimum(iy + 1, n - 1)

    f00 = plsc.load_gather(tab_v, [iy, ix])
    f01 = plsc.load_gather(tab_v, [iy, ixn])
    f10 = plsc.load_gather(tab_v, [iyn, ix])
    f11 = plsc.load_gather(tab_v, [iyn, ixn])

    r0 = f00 + alpha * (f01 - f00)
    r1 = f10 + alpha * (f11 - f10)
    return r0 + beta * (r1 - r0)


def _make_sc_kernel(n_total, chunk):
    per_w = n_total // NUM_WORKERS
    n_chunks = per_w // chunk
    vecs = chunk // LANES

    mesh = plsc.VectorSubcoreMesh(core_axis_name="c", subcore_axis_name="s")

    @functools.partial(
        pl.kernel,
        mesh=mesh,
        out_type=jax.ShapeDtypeStruct((n_total,), jnp.float32),
        compiler_params=pltpu.CompilerParams(needs_layout_passes=False),
        scratch_types=[
            pltpu.VMEM((N_BREAKS, N_BREAKS), jnp.float32),
            pltpu.VMEM((chunk,), jnp.float32),
            pltpu.VMEM((chunk,), jnp.float32),
            pltpu.VMEM((chunk,), jnp.float32),
        ],
    )
    def sc_kernel(x_hbm, y_hbm, tab_hbm, out_hbm, tab_v, x_v, y_v, o_v):
        wid = lax.axis_index("s") * NUM_CORES + lax.axis_index("c")
        pltpu.sync_copy(tab_hbm, tab_v)
        base = wid * per_w
        for c in range(n_chunks):
            off = base + c * chunk
            pltpu.sync_copy(x_hbm.at[pl.ds(off, chunk)], x_v)
            pltpu.sync_copy(y_hbm.at[pl.ds(off, chunk)], y_v)

            def vec_body(i, carry):
                s = i * LANES
                o_v[pl.ds(s, LANES)] = _interp_vec(
                    x_v[pl.ds(s, LANES)], y_v[pl.ds(s, LANES)], tab_v
                )
                return carry

            lax.fori_loop(0, vecs, vec_body, 0, unroll=8)
            pltpu.sync_copy(o_v, out_hbm.at[pl.ds(off, chunk)])

    return sc_kernel


def kernel(x, y, inv_softplus_step_values):
    table = _build_table(inv_softplus_step_values)
    xf = x.reshape(-1)
    yf = y.reshape(-1)
    n_total = xf.shape[0]
    chunk = 12800
    out = _make_sc_kernel(n_total, chunk)(xf, yf, table)
    return out.reshape(x.shape)


# speedup vs baseline: 1.5233x; 1.5233x over previous
import functools

import jax
import jax.numpy as jnp
from jax import lax
from jax.experimental import pallas as pl
from jax.experimental.pallas import tpu as pltpu
from jax.experimental.pallas import tpu_sc as plsc

N_BREAKS = 128
INPUT_RANGE = 8.0
DX = 2.0 * INPUT_RANGE / (N_BREAKS - 1)

NUM_CORES = 2
NUM_SUBCORES = 16
NUM_WORKERS = NUM_CORES * NUM_SUBCORES
LANES = 16


def _table_body(w_ref, out_ref):
    n = N_BREAKS
    R = INPUT_RANGE
    dx = DX
    f32 = jnp.float32
    hi = lax.Precision.HIGHEST

    w = w_ref[...]
    ri = lax.broadcasted_iota(jnp.int32, (n, n), 0)
    ci = lax.broadcasted_iota(jnp.int32, (n, n), 1)

    sp = jnp.maximum(w, 0.0) + jnp.log(1.0 + jnp.exp(-jnp.abs(w)))
    cell = sp * (dx * dx)

    U = (ri <= ci).astype(f32)
    L = (ci <= ri).astype(f32)
    cx = lax.dot(cell, U, precision=hi)
    cxy = lax.dot(L, cx, precision=hi)

    D1 = (ci == ri - 1).astype(f32)
    M1 = (ri == ci - 1).astype(f32)
    interior = lax.dot(D1, lax.dot(cxy, M1, precision=hi), precision=hi)

    row0 = jnp.sum(jnp.where(ri == 0, cx, 0.0), axis=0, keepdims=True) * (1.0 / dx)
    row0s = lax.dot(jnp.broadcast_to(row0, (n, n)), M1, precision=hi)
    cy = lax.dot(L, cell, precision=hi)
    col0 = jnp.sum(jnp.where(ci == 0, cy, 0.0), axis=1, keepdims=True) * (1.0 / dx)
    col0s = lax.dot(D1, jnp.broadcast_to(col0, (n, n)), precision=hi)

    res = jnp.where(ri == 0, row0s, jnp.where(ci == 0, col0s, interior)) - R

    tr = jnp.sum(jnp.where((ri == n - 1) & (ci == n - 1), res, 0.0)) + R
    ok = tr > 1e-10
    scale = (2.0 * R) / jnp.where(ok, tr, 1.0)
    res = jnp.where(ok, -R + (res + R) * scale, res)

    a = res + R
    for k in (1, 2, 4, 8, 16, 32, 64):
        Dk = (ci == ri - k).astype(f32)
        a = jnp.maximum(a, lax.dot(Dk, a, precision=hi))
    for k in (1, 2, 4, 8, 16, 32, 64):
        Mk = (ri == ci - k).astype(f32)
        a = jnp.maximum(a, lax.dot(a, Mk, precision=hi))
    out_ref[...] = a - R


def _build_table(w):
    return pl.pallas_call(
        _table_body,
        out_shape=jax.ShapeDtypeStruct((N_BREAKS, N_BREAKS), jnp.float32),
    )(w)


def _interp_vec(xx, yy, tab_v):
    n = N_BREAKS
    inv_dx = float(1.0 / DX)

    xn = (xx + INPUT_RANGE) * inv_dx
    yn = (yy + INPUT_RANGE) * inv_dx
    xc = jnp.clip(xn, 0.0, float(n - 1))
    yc = jnp.clip(yn, 0.0, float(n - 1))
    ix = xc.astype(jnp.int32)
    iy = yc.astype(jnp.int32)
    alpha = jnp.clip(xn - ix.astype(jnp.float32), 0.0, 1.0)
    beta = jnp.clip(yn - iy.astype(jnp.float32), 0.0, 1.0)
    ixn = jnp.minimum(ix + 1, n - 1)
    iyn = jnp.minimum(iy + 1, n - 1)

    f00 = plsc.load_gather(tab_v, [iy, ix])
    f01 = plsc.load_gather(tab_v, [iy, ixn])
    f10 = plsc.load_gather(tab_v, [iyn, ix])
    f11 = plsc.load_gather(tab_v, [iyn, ixn])

    r0 = f00 + alpha * (f01 - f00)
    r1 = f10 + alpha * (f11 - f10)
    return r0 + beta * (r1 - r0)


def _make_sc_kernel(n_total, chunk):
    per_w = n_total // NUM_WORKERS
    n_chunks = per_w // chunk
    vecs = chunk // LANES

    mesh = plsc.VectorSubcoreMesh(core_axis_name="c", subcore_axis_name="s")

    @functools.partial(
        pl.kernel,
        mesh=mesh,
        out_type=jax.ShapeDtypeStruct((n_total,), jnp.float32),
        compiler_params=pltpu.CompilerParams(needs_layout_passes=False),
        scratch_types=[
            pltpu.VMEM((N_BREAKS, N_BREAKS), jnp.float32),
            pltpu.VMEM((chunk,), jnp.float32),
            pltpu.VMEM((chunk,), jnp.float32),
            pltpu.VMEM((chunk,), jnp.float32),
        ],
    )
    def sc_kernel(x_hbm, y_hbm, tab_hbm, out_hbm, tab_v, x_v, y_v, o_v):
        wid = lax.axis_index("s") * NUM_CORES + lax.axis_index("c")
        pltpu.sync_copy(tab_hbm, tab_v)
        base = wid * per_w
        for c in range(n_chunks):
            off = base + c * chunk
            pltpu.sync_copy(x_hbm.at[pl.ds(off, chunk)], x_v)
            pltpu.sync_copy(y_hbm.at[pl.ds(off, chunk)], y_v)

            @plsc.parallel_loop(0, vecs, 1, unroll=4)
            def vec_body(i):
                s = i * LANES
                o_v[pl.ds(s, LANES)] = _interp_vec(
                    x_v[pl.ds(s, LANES)], y_v[pl.ds(s, LANES)], tab_v
                )
            pltpu.sync_copy(o_v, out_hbm.at[pl.ds(off, chunk)])

    return sc_kernel


def kernel(x, y, inv_softplus_step_values):
    table = _build_table(inv_softplus_step_values)
    xf = x.reshape(-1)
    yf = y.reshape(-1)
    n_total = xf.shape[0]
    chunk = 12800
    out = _make_sc_kernel(n_total, chunk)(xf, yf, table)
    return out.reshape(x.shape)


# speedup vs baseline: 1.6727x; 1.0981x over previous
import functools

import jax
import jax.numpy as jnp
from jax import lax
from jax.experimental import pallas as pl
from jax.experimental.pallas import tpu as pltpu
from jax.experimental.pallas import tpu_sc as plsc

N_BREAKS = 128
INPUT_RANGE = 8.0
DX = 2.0 * INPUT_RANGE / (N_BREAKS - 1)

NUM_CORES = 2
NUM_SUBCORES = 16
NUM_WORKERS = NUM_CORES * NUM_SUBCORES
LANES = 16


def _table_body(w_ref, out_ref):
    n = N_BREAKS
    R = INPUT_RANGE
    dx = DX
    f32 = jnp.float32
    hi = lax.Precision.HIGHEST

    w = w_ref[...]
    ri = lax.broadcasted_iota(jnp.int32, (n, n), 0)
    ci = lax.broadcasted_iota(jnp.int32, (n, n), 1)

    sp = jnp.maximum(w, 0.0) + jnp.log(1.0 + jnp.exp(-jnp.abs(w)))
    cell = sp * (dx * dx)

    U = (ri <= ci).astype(f32)
    L = (ci <= ri).astype(f32)
    cx = lax.dot(cell, U, precision=hi)
    cxy = lax.dot(L, cx, precision=hi)

    D1 = (ci == ri - 1).astype(f32)
    M1 = (ri == ci - 1).astype(f32)
    interior = lax.dot(D1, lax.dot(cxy, M1, precision=hi), precision=hi)

    row0 = jnp.sum(jnp.where(ri == 0, cx, 0.0), axis=0, keepdims=True) * (1.0 / dx)
    row0s = lax.dot(jnp.broadcast_to(row0, (n, n)), M1, precision=hi)
    cy = lax.dot(L, cell, precision=hi)
    col0 = jnp.sum(jnp.where(ci == 0, cy, 0.0), axis=1, keepdims=True) * (1.0 / dx)
    col0s = lax.dot(D1, jnp.broadcast_to(col0, (n, n)), precision=hi)

    res = jnp.where(ri == 0, row0s, jnp.where(ci == 0, col0s, interior)) - R

    tr = jnp.sum(jnp.where((ri == n - 1) & (ci == n - 1), res, 0.0)) + R
    ok = tr > 1e-10
    scale = (2.0 * R) / jnp.where(ok, tr, 1.0)
    res = jnp.where(ok, -R + (res + R) * scale, res)

    a = res + R
    for k in (1, 2, 4, 8, 16, 32, 64):
        Dk = (ci == ri - k).astype(f32)
        a = jnp.maximum(a, lax.dot(Dk, a, precision=hi))
    for k in (1, 2, 4, 8, 16, 32, 64):
        Mk = (ri == ci - k).astype(f32)
        a = jnp.maximum(a, lax.dot(a, Mk, precision=hi))
    out_ref[...] = a - R


def _build_table(w):
    return pl.pallas_call(
        _table_body,
        out_shape=jax.ShapeDtypeStruct((N_BREAKS, N_BREAKS), jnp.float32),
    )(w)


def _interp_vec(xx, yy, tab_v):
    n = N_BREAKS
    inv_dx = float(1.0 / DX)

    xn = (xx + INPUT_RANGE) * inv_dx
    yn = (yy + INPUT_RANGE) * inv_dx
    xc = jnp.clip(xn, 0.0, float(n - 1))
    yc = jnp.clip(yn, 0.0, float(n - 1))
    ix = xc.astype(jnp.int32)
    iy = yc.astype(jnp.int32)
    alpha = jnp.clip(xn - ix.astype(jnp.float32), 0.0, 1.0)
    beta = jnp.clip(yn - iy.astype(jnp.float32), 0.0, 1.0)
    ixn = jnp.minimum(ix + 1, n - 1)
    iyn = jnp.minimum(iy + 1, n - 1)

    f00 = plsc.load_gather(tab_v, [iy, ix])
    f01 = plsc.load_gather(tab_v, [iy, ixn])
    f10 = plsc.load_gather(tab_v, [iyn, ix])
    f11 = plsc.load_gather(tab_v, [iyn, ixn])

    r0 = f00 + alpha * (f01 - f00)
    r1 = f10 + alpha * (f11 - f10)
    return r0 + beta * (r1 - r0)


def _make_sc_kernel(n_total, chunk):
    per_w = n_total // NUM_WORKERS
    n_chunks = per_w // chunk
    vecs = chunk // LANES

    mesh = plsc.VectorSubcoreMesh(core_axis_name="c", subcore_axis_name="s")

    @functools.partial(
        pl.kernel,
        mesh=mesh,
        out_type=jax.ShapeDtypeStruct((n_total,), jnp.float32),
        compiler_params=pltpu.CompilerParams(needs_layout_passes=False),
        scratch_types=[
            pltpu.VMEM((N_BREAKS, N_BREAKS), jnp.float32),
            [pltpu.VMEM((chunk,), jnp.float32)] * 2,
            [pltpu.VMEM((chunk,), jnp.float32)] * 2,
            [pltpu.VMEM((chunk,), jnp.float32)] * 2,
            [pltpu.SemaphoreType.DMA] * 6,
        ],
    )
    def sc_kernel(x_hbm, y_hbm, tab_hbm, out_hbm, tab_v, x_v, y_v, o_v, sems):
        wid = lax.axis_index("s") * NUM_CORES + lax.axis_index("c")
        pltpu.sync_copy(tab_hbm, tab_v)
        base = wid * per_w
        xcp = [None, None]
        ycp = [None, None]
        ocp = [None, None]
        xcp[0] = pltpu.async_copy(x_hbm.at[pl.ds(base, chunk)], x_v[0], sems[0])
        ycp[0] = pltpu.async_copy(y_hbm.at[pl.ds(base, chunk)], y_v[0], sems[2])
        for c in range(n_chunks):
            b = c & 1
            if c + 1 < n_chunks:
                nb = 1 - b
                noff = base + (c + 1) * chunk
                xcp[nb] = pltpu.async_copy(
                    x_hbm.at[pl.ds(noff, chunk)], x_v[nb], sems[nb])
                ycp[nb] = pltpu.async_copy(
                    y_hbm.at[pl.ds(noff, chunk)], y_v[nb], sems[2 + nb])
            xcp[b].wait()
            ycp[b].wait()
            if ocp[b] is not None:
                ocp[b].wait()
            xb, yb, ob = x_v[b], y_v[b], o_v[b]

            @plsc.parallel_loop(0, vecs, 1, unroll=4)
            def vec_body(i):
                s = i * LANES
                ob[pl.ds(s, LANES)] = _interp_vec(
                    xb[pl.ds(s, LANES)], yb[pl.ds(s, LANES)], tab_v
                )
            ocp[b] = pltpu.async_copy(
                ob, out_hbm.at[pl.ds(base + c * chunk, chunk)], sems[4 + b])
        ocp[0].wait()
        ocp[1].wait()

    return sc_kernel


def kernel(x, y, inv_softplus_step_values):
    table = _build_table(inv_softplus_step_values)
    xf = x.reshape(-1)
    yf = y.reshape(-1)
    n_total = xf.shape[0]
    chunk = 12800
    out = _make_sc_kernel(n_total, chunk)(xf, yf, table)
    return out.reshape(x.shape)
